# baseline (device time: 79141 ns/iter reference)
import os

import jax
import jax.numpy as jnp
from jax import lax
from jax.experimental import pallas as pl
from jax.experimental.pallas import tpu as pltpu

N_DEV = 4

_KMODE = os.environ.get("KMODE", "full")


def kernel(x, w_mat):
    m_per, k = x.shape
    _, n_per = w_mat.shape
    m_total = N_DEV * m_per
    m_half = m_per // 2
    m_q = m_per // 4
    do_gemm = _KMODE in ("full", "gemm")

    if _KMODE == "gemm":
        def gemm_body(x_ref, w_ref, out_ref):
            for c in range(N_DEV):
                out_ref[pl.ds(c * m_per, m_per), :] = jnp.dot(
                    x_ref[...], w_ref[...],
                    preferred_element_type=jnp.float32,
                )
        return pl.pallas_call(
            gemm_body,
            out_shape=jax.ShapeDtypeStruct((m_total, n_per), jnp.float32),
            in_specs=[pl.BlockSpec(memory_space=pltpu.VMEM),
                      pl.BlockSpec(memory_space=pltpu.VMEM)],
            out_specs=pl.BlockSpec(memory_space=pltpu.VMEM),
        )(x, w_mat)

    probe = _KMODE if _KMODE in ("bar", "send1", "p0", "bar_any") else None

    if probe == "bar_any":
        def bar_any_body(x_ref, w_ref, out_ref):
            my = lax.axis_index("i")
            left = lax.rem(my - 1 + N_DEV, N_DEV)
            right = lax.rem(my + 1, N_DEV)
            if os.environ.get("NOBAR") != "1":
                barrier_sem = pltpu.get_barrier_semaphore()
                for nbr in [left, right]:
                    pl.semaphore_signal(
                        barrier_sem, inc=1,
                        device_id=(nbr,), device_id_type=pl.DeviceIdType.MESH,
                    )
                pl.semaphore_wait(barrier_sem, 2)
        return pl.pallas_call(
            bar_any_body,
            out_shape=jax.ShapeDtypeStruct((m_total, n_per), jnp.float32),
            in_specs=[pl.BlockSpec(memory_space=pl.ANY),
                      pl.BlockSpec(memory_space=pl.ANY)],
            out_specs=pl.BlockSpec(memory_space=pl.ANY),
            **(
                {}
                if os.environ.get("NOBAR") == "1"
                else {"compiler_params": pltpu.CompilerParams(collective_id=0)}
            ),
        )(x, w_mat)

    if probe is not None:
        def probe_body(x_ref, w_ref, out_ref, gath_ref, send_sems, recv_sems):
            my = lax.axis_index("i")
            left = lax.rem(my - 1 + N_DEV, N_DEV)
            right = lax.rem(my + 1, N_DEV)
            my_row = my * m_per
            left_row = left * m_per
            right_row = right * m_per

            barrier_sem = pltpu.get_barrier_semaphore()
            for nbr in [left, right]:
                pl.semaphore_signal(
                    barrier_sem, inc=1,
                    device_id=(nbr,), device_id_type=pl.DeviceIdType.MESH,
                )
            pl.semaphore_wait(barrier_sem, 2)
            if probe == "bar":
                return
            gath_ref[pl.ds(my_row, m_per), :] = x_ref[...]
            s_r0 = pltpu.make_async_remote_copy(
                src_ref=gath_ref.at[pl.ds(my_row, m_per), :],
                dst_ref=gath_ref.at[pl.ds(my_row, m_per), :],
                send_sem=send_sems.at[0], recv_sem=recv_sems.at[0],
                device_id=(right,), device_id_type=pl.DeviceIdType.MESH,
            )
            s_r0.start()
            if probe == "p0":
                s_l0 = pltpu.make_async_remote_copy(
                    src_ref=gath_ref.at[pl.ds(my_row, m_per), :],
                    dst_ref=gath_ref.at[pl.ds(my_row, m_per), :],
                    send_sem=send_sems.at[1], recv_sem=recv_sems.at[1],
                    device_id=(left,), device_id_type=pl.DeviceIdType.MESH,
                )
                s_l0.start()
            r_l0 = pltpu.make_async_remote_copy(
                src_ref=gath_ref.at[pl.ds(left_row, m_per), :],
                dst_ref=gath_ref.at[pl.ds(left_row, m_per), :],
                send_sem=send_sems.at[0], recv_sem=recv_sems.at[0],
                device_id=(left,), device_id_type=pl.DeviceIdType.MESH,
            )
            r_l0.wait_recv()
            if probe == "p0":
                r_r0 = pltpu.make_async_remote_copy(
                    src_ref=gath_ref.at[pl.ds(right_row, m_per), :],
                    dst_ref=gath_ref.at[pl.ds(right_row, m_per), :],
                    send_sem=send_sems.at[1], recv_sem=recv_sems.at[1],
                    device_id=(right,), device_id_type=pl.DeviceIdType.MESH,
                )
                r_r0.wait_recv()
                s_l0.wait_send()
            s_r0.wait_send()

        return pl.pallas_call(
            probe_body,
            out_shape=jax.ShapeDtypeStruct((m_total, n_per), jnp.float32),
            in_specs=[pl.BlockSpec(memory_space=pltpu.VMEM),
                      pl.BlockSpec(memory_space=pltpu.VMEM)],
            out_specs=pl.BlockSpec(memory_space=pltpu.VMEM),
            scratch_shapes=[
                pltpu.VMEM((m_total, k), jnp.float32),
                pltpu.SemaphoreType.DMA((2,)),
                pltpu.SemaphoreType.DMA((2,)),
            ],
            compiler_params=pltpu.CompilerParams(collective_id=0),
        )(x, w_mat)

    def body(x_ref, w_ref, out_ref, gath_ref, w_v, y_ref,
             send_sems, recv_sems, loc_sems, out_sems):
        my = lax.axis_index("i")
        left = lax.rem(my - 1 + N_DEV, N_DEV)
        right = lax.rem(my + 1, N_DEV)
        opp = lax.rem(my + 2, N_DEV)

        my_row = my * m_per
        left_row = left * m_per
        right_row = right * m_per
        opp_row = opp * m_per

        def rc(src_rows, src_n, dst_rows, dst_n, sem_idx, nbr, src=None):
            src = gath_ref if src is None else src
            return pltpu.make_async_remote_copy(
                src_ref=src.at[pl.ds(src_rows, src_n), :],
                dst_ref=gath_ref.at[pl.ds(dst_rows, dst_n), :],
                send_sem=send_sems.at[sem_idx],
                recv_sem=recv_sems.at[sem_idx],
                device_id=(nbr,), device_id_type=pl.DeviceIdType.MESH,
            )

        cx = pltpu.make_async_copy(
            x_ref, gath_ref.at[pl.ds(my_row, m_per), :], loc_sems.at[0]
        )
        cx.start()
        cw = pltpu.make_async_copy(w_ref, w_v, loc_sems.at[1])
        cw.start()

        barrier_sem = pltpu.get_barrier_semaphore()
        for nbr in [left, right]:
            pl.semaphore_signal(
                barrier_sem, inc=1,
                device_id=(nbr,), device_id_type=pl.DeviceIdType.MESH,
            )
        pl.semaphore_wait(barrier_sem, 2)

        cx.wait()
        s_r0a = rc(my_row, m_half, my_row, m_half, 0, right)
        s_r0a.start()
        s_l0b = rc(my_row + m_half, m_half, my_row + m_half, m_half, 2, left)
        s_l0b.start()
        s_r0b = rc(my_row + m_half, m_half, my_row + m_half, m_half, 1, right)
        s_r0b.start()
        s_l0a = rc(my_row, m_half, my_row, m_half, 3, left)
        s_l0a.start()

        cw.wait()
        if do_gemm:
            y_ref[pl.ds(my_row, m_per), :] = jnp.dot(
                gath_ref[pl.ds(my_row, m_per), :], w_v[...],
                preferred_element_type=jnp.float32,
            )
        co_my = pltpu.make_async_copy(
            y_ref.at[pl.ds(my_row, m_per), :],
            out_ref.at[pl.ds(my_row, m_per), :],
            out_sems.at[0],
        )
        co_my.start()

        rc(left_row, m_half, left_row, m_half, 0, left).wait_recv()
        s_r1a = rc(left_row, m_q, left_row, m_q, 4, right)
        s_r1a.start()
        s_r1b = rc(left_row + m_q, m_q, left_row + m_q, m_q, 5, right)
        s_r1b.start()

        rc(right_row + m_half, m_half, right_row + m_half, m_half,
           2, right).wait_recv()
        s_l1a = rc(right_row + m_half, m_q, right_row + m_half, m_q, 6, left)
        s_l1a.start()
        s_l1b = rc(right_row + m_half + m_q, m_q,
                   right_row + m_half + m_q, m_q, 7, left)
        s_l1b.start()

        if do_gemm:
            y_ref[pl.ds(left_row, m_half), :] = jnp.dot(
                gath_ref[pl.ds(left_row, m_half), :], w_v[...],
                preferred_element_type=jnp.float32,
            )
            y_ref[pl.ds(right_row + m_half, m_half), :] = jnp.dot(
                gath_ref[pl.ds(right_row + m_half, m_half), :], w_v[...],
                preferred_element_type=jnp.float32,
            )

        rc(left_row + m_half, m_half, left_row + m_half, m_half,
           1, left).wait_recv()
        if do_gemm:
            y_ref[pl.ds(left_row + m_half, m_half), :] = jnp.dot(
                gath_ref[pl.ds(left_row + m_half, m_half), :], w_v[...],
                preferred_element_type=jnp.float32,
            )
        co_left = pltpu.make_async_copy(
            y_ref.at[pl.ds(left_row, m_per), :],
            out_ref.at[pl.ds(left_row, m_per), :],
            out_sems.at[1],
        )
        co_left.start()

        rc(right_row, m_half, right_row, m_half, 3, right).wait_recv()
        if do_gemm:
            y_ref[pl.ds(right_row, m_half), :] = jnp.dot(
                gath_ref[pl.ds(right_row, m_half), :], w_v[...],
                preferred_element_type=jnp.float32,
            )
        co_right = pltpu.make_async_copy(
            y_ref.at[pl.ds(right_row, m_per), :],
            out_ref.at[pl.ds(right_row, m_per), :],
            out_sems.at[2],
        )
        co_right.start()

        out_copies = [co_my, co_left, co_right]
        for strip, sem_idx, nbr in ((0, 4, left), (2, 6, right),
                                    (1, 5, left), (3, 7, right)):
            row = opp_row + strip * m_q
            rc(row, m_q, row, m_q, sem_idx, nbr).wait_recv()
            if do_gemm:
                y_ref[pl.ds(row, m_q), :] = jnp.dot(
                    gath_ref[pl.ds(row, m_q), :], w_v[...],
                    preferred_element_type=jnp.float32,
                )
            co = pltpu.make_async_copy(
                y_ref.at[pl.ds(row, m_q), :],
                out_ref.at[pl.ds(row, m_q), :],
                out_sems.at[3 + strip],
            )
            co.start()
            out_copies.append(co)

        for s in (s_r0a, s_r0b, s_l0a, s_l0b, s_r1a, s_r1b, s_l1a, s_l1b):
            s.wait_send()
        for co in out_copies:
            co.wait()

    return pl.pallas_call(
        body,
        out_shape=jax.ShapeDtypeStruct((m_total, n_per), jnp.float32),
        in_specs=[
            pl.BlockSpec(memory_space=pl.ANY),
            pl.BlockSpec(memory_space=pl.ANY),
        ],
        out_specs=pl.BlockSpec(memory_space=pl.ANY),
        scratch_shapes=[
            pltpu.VMEM((m_total, k), jnp.float32),
            pltpu.VMEM((k, n_per), jnp.float32),
            pltpu.VMEM((m_total, n_per), jnp.float32),
            pltpu.SemaphoreType.DMA((8,)),
            pltpu.SemaphoreType.DMA((8,)),
            pltpu.SemaphoreType.DMA((2,)),
            pltpu.SemaphoreType.DMA((7,)),
        ],
        compiler_params=pltpu.CompilerParams(collective_id=0),
    )(x, w_mat)
